# Initial kernel scaffold; baseline (speedup 1.0000x reference)
import jax
import jax.numpy as jnp
from jax.experimental import pallas as pl


def kernel(x, W):
    raise NotImplementedError("write your pallas kernel here")


# speedup vs baseline: 1.0152x; 1.0152x over previous
import jax
import jax.numpy as jnp
from jax import lax
from jax.experimental import pallas as pl
from jax.experimental.pallas import tpu as pltpu
from jax.experimental.pallas import tpu_sc as plsc

EMB = 1000
NC, NS = 2, 16
NW = NC * NS
B_TOTAL = 1024 * 50
B_PER_W = B_TOTAL // NW
CHUNK = 64
N_CHUNKS = B_PER_W // CHUNK


def _gather_body(x_hbm, w_hbm, out_hbm, idx_v, rows_v, sem):
    wid = lax.axis_index("s") * NC + lax.axis_index("c")
    pltpu.sync_copy(x_hbm.at[wid], idx_v)
    base = wid * B_PER_W

    def chunk_body(c, carry):
        pltpu.async_copy(w_hbm.at[idx_v.at[c]], rows_v, sem).wait()
        pltpu.sync_copy(rows_v, out_hbm.at[pl.ds(base + c * CHUNK, CHUNK)])
        return carry

    lax.fori_loop(0, N_CHUNKS, chunk_body, 0)


def kernel(x, W):
    xr = x.reshape(NW, N_CHUNKS, CHUNK)
    mesh = plsc.VectorSubcoreMesh(core_axis_name="c", subcore_axis_name="s")
    out = pl.kernel(
        _gather_body,
        out_type=jax.ShapeDtypeStruct((B_TOTAL, EMB), jnp.float32),
        mesh=mesh,
        scratch_types=[
            pltpu.VMEM((N_CHUNKS, CHUNK), jnp.int32),
            pltpu.VMEM((CHUNK, EMB), jnp.float32),
            pltpu.SemaphoreType.DMA,
        ],
        compiler_params=pltpu.CompilerParams(use_tc_tiling_on_sc=False),
    )(xr, W)
    return out.reshape(x.shape[0], x.shape[1], EMB)


# speedup vs baseline: 1.0347x; 1.0193x over previous
import jax
import jax.numpy as jnp
from jax import lax
from jax.experimental import pallas as pl
from jax.experimental.pallas import tpu as pltpu
from jax.experimental.pallas import tpu_sc as plsc

EMB = 1000
NC, NS = 2, 16
NW = NC * NS
B_TOTAL = 1024 * 50
B_PER_W = B_TOTAL // NW
CHUNK = 40
N_CHUNKS = B_PER_W // CHUNK


def _gather_body(x_hbm, w_hbm, out_hbm, idx_v, rows_v, sem0, sem1):
    wid = lax.axis_index("s") * NC + lax.axis_index("c")
    pltpu.sync_copy(x_hbm.at[wid], idx_v)
    base = wid * B_PER_W
    sems = (sem0, sem1)

    def _start(c, b):
        pltpu.async_copy(w_hbm.at[idx_v.at[c]], rows_v.at[b], sems[b])

    def _wait(b):
        pltpu.make_async_copy(w_hbm.at[idx_v.at[0]], rows_v.at[b], sems[b]).wait()

    def _drain(c, b):
        _wait(b)
        pltpu.sync_copy(rows_v.at[b], out_hbm.at[pl.ds(base + c * CHUNK, CHUNK)])

    for b in range(2):
        _start(b, b)

    def pair_body(i, carry):
        c0 = 2 * i
        for b in range(2):
            _drain(c0 + b, b)
            _start(c0 + b + 2, b)
        return carry

    lax.fori_loop(0, N_CHUNKS // 2 - 1, pair_body, 0)
    for b in range(2):
        _drain(N_CHUNKS - 2 + b, b)


def kernel(x, W):
    xr = x.reshape(NW, N_CHUNKS, CHUNK)
    mesh = plsc.VectorSubcoreMesh(core_axis_name="c", subcore_axis_name="s")
    out = pl.kernel(
        _gather_body,
        out_type=jax.ShapeDtypeStruct((B_TOTAL, EMB), jnp.float32),
        mesh=mesh,
        scratch_types=[
            pltpu.VMEM((N_CHUNKS, CHUNK), jnp.int32),
            pltpu.VMEM((2, CHUNK, EMB), jnp.float32),
            pltpu.SemaphoreType.DMA,
            pltpu.SemaphoreType.DMA,
        ],
        compiler_params=pltpu.CompilerParams(use_tc_tiling_on_sc=False),
    )(xr, W)
    return out.reshape(x.shape[0], x.shape[1], EMB)


# speedup vs baseline: 1.0355x; 1.0007x over previous
import jax
import jax.numpy as jnp
from jax import lax
from jax.experimental import pallas as pl
from jax.experimental.pallas import tpu as pltpu
from jax.experimental.pallas import tpu_sc as plsc

EMB = 1000
BATCH = 1024
HIST = 50
NC, NS = 2, 16
NW = NC * NS
B_PER_W = BATCH // NW


def _gather_body(x_hbm, w_hbm, out_hbm, idx_v, rows_v, sem0, sem1):
    wid = lax.axis_index("s") * NC + lax.axis_index("c")
    pltpu.sync_copy(x_hbm.at[wid], idx_v)
    base = wid * B_PER_W
    sems = (sem0, sem1)

    def _start(c, b):
        pltpu.async_copy(w_hbm.at[idx_v.at[c]], rows_v.at[b], sems[b])

    def _wait(b):
        pltpu.make_async_copy(w_hbm.at[idx_v.at[0]], rows_v.at[b], sems[b]).wait()

    def _drain(c, b):
        _wait(b)
        pltpu.sync_copy(rows_v.at[b], out_hbm.at[base + c])

    for b in range(2):
        _start(b, b)

    def pair_body(i, carry):
        c0 = 2 * i
        for b in range(2):
            _drain(c0 + b, b)
            _start(c0 + b + 2, b)
        return carry

    lax.fori_loop(0, B_PER_W // 2 - 1, pair_body, 0)
    for b in range(2):
        _drain(B_PER_W - 2 + b, b)


def kernel(x, W):
    xr = x.reshape(NW, B_PER_W, HIST)
    mesh = plsc.VectorSubcoreMesh(core_axis_name="c", subcore_axis_name="s")
    out = pl.kernel(
        _gather_body,
        out_type=jax.ShapeDtypeStruct((BATCH, HIST, EMB), jnp.float32),
        mesh=mesh,
        scratch_types=[
            pltpu.VMEM((B_PER_W, HIST), jnp.int32),
            pltpu.VMEM((2, HIST, EMB), jnp.float32),
            pltpu.SemaphoreType.DMA,
            pltpu.SemaphoreType.DMA,
        ],
        compiler_params=pltpu.CompilerParams(use_tc_tiling_on_sc=False),
    )(xr, W)
    return out
